# baseline (speedup 1.0000x reference)
import functools

import jax
import jax.numpy as jnp
from jax import lax
from jax.experimental import pallas as pl
from jax.experimental.pallas import tpu as pltpu
from jax.experimental.pallas import tpu_sc as plsc

_B = 8
_N1 = 1024
_S1 = 512
_K1 = 32
_R1 = 0.2
_S2 = 256
_K2 = 64
_R2 = 0.4
_H = 64
_EPS = 1e-5
_F32 = jnp.float32
_I32 = jnp.int32


def _rne_bf16(x):
    bits = plsc.bitcast(x, _I32)
    half = jnp.full((16,), 0x7FFF, _I32)
    one = jnp.full((16,), 1, _I32)
    lsb = lax.shift_right_logical(bits, jnp.full((16,), 16, _I32)) & one
    r = (bits + half + lsb) & jnp.full((16,), -65536, _I32)
    return plsc.bitcast(r, _F32)


def _fps_phase(px, py, pz, distv, N, S, store_cb):
    CH = N // 16
    for j in range(CH):
        distv[pl.ds(j * 16, 16)] = jnp.full((16,), 1e10, _F32)
    iot = lax.iota(_I32, 16)
    lane0 = iot == 0

    def step(i, f):
        fv = jnp.full((16,), f, _I32)
        iv = jnp.full((16,), i, _I32)
        cx = plsc.load_gather(px, [fv])
        cy = plsc.load_gather(py, [fv])
        cz = plsc.load_gather(pz, [fv])
        store_cb(iv, lane0, cx, cy, cz)
        bestv = jnp.full((16,), -1.0, _F32)
        besti = jnp.zeros((16,), _I32)
        for j in range(CH):
            dx = px[pl.ds(j * 16, 16)] - cx
            dy = py[pl.ds(j * 16, 16)] - cy
            dz = pz[pl.ds(j * 16, 16)] - cz
            d = (dx * dx + dy * dy) + dz * dz
            dn = jnp.minimum(distv[pl.ds(j * 16, 16)], d)
            distv[pl.ds(j * 16, 16)] = dn
            upd = dn > bestv
            bestv = jnp.where(upd, dn, bestv)
            besti = jnp.where(upd, iot + (j * 16), besti)
        m = jnp.max(bestv)
        cand = jnp.where(bestv == m, besti, N)
        return jnp.min(cand)

    lax.fori_loop(0, S, step, jnp.int32(0))


def _make_stage1():
    N, S, K, r2 = _N1, _S1, _K1, _R1 * _R1
    CH = N // 16
    S4 = S // 4
    PK = K // 16
    mesh = plsc.VectorSubcoreMesh(core_axis_name="c", subcore_axis_name="s")
    out_type = [
        jax.ShapeDtypeStruct((_B * 3 * _S1,), _F32),
        jax.ShapeDtypeStruct((_B * 3 * _S2,), _F32),
        jax.ShapeDtypeStruct((_B * _S2 * 4,), _F32),
        jax.ShapeDtypeStruct((_B * S * K * 4,), _F32),
    ]
    scratch = [
        pltpu.VMEM((N,), _F32),
        pltpu.VMEM((N,), _F32),
        pltpu.VMEM((N,), _F32),
        pltpu.VMEM((N,), _F32),
        pltpu.VMEM((_S1,), _F32),
        pltpu.VMEM((_S1,), _F32),
        pltpu.VMEM((_S1,), _F32),
        pltpu.VMEM((_S2,), _F32),
        pltpu.VMEM((_S2,), _F32),
        pltpu.VMEM((_S2,), _F32),
        pltpu.VMEM((_S2 * 4,), _F32),
        pltpu.VMEM((N,), _F32),
        pltpu.VMEM((N,), _F32),
        pltpu.VMEM((N,), _F32),
        pltpu.VMEM((N,), _F32),
        pltpu.VMEM((S4,), _F32),
        pltpu.VMEM((S4,), _F32),
        pltpu.VMEM((S4,), _F32),
        pltpu.VMEM((N + 16,), _I32),
        pltpu.VMEM((S4 * K * 4,), _F32),
    ]

    def body(pts_hbm, out_c1, out_c2, out4f, out4b,
             px, py, pz, distv, sx, sy, sz, tx, ty, tz, stag4f,
             pxb, pyb, pzb, s2v, cxb, cyb, czb, ib, stag4b):
        wid = lax.axis_index("c") * 16 + lax.axis_index("s")

        @pl.when(wid % 4 == 0)
        def _():
            b = wid // 4
            pltpu.sync_copy(pts_hbm.at[pl.ds((b * 3 + 0) * N, N)], px)
            pltpu.sync_copy(pts_hbm.at[pl.ds((b * 3 + 1) * N, N)], py)
            pltpu.sync_copy(pts_hbm.at[pl.ds((b * 3 + 2) * N, N)], pz)

            def store1(iv, lane0, cx, cy, cz):
                plsc.store_scatter(sx, [iv], cx, mask=lane0)
                plsc.store_scatter(sy, [iv], cy, mask=lane0)
                plsc.store_scatter(sz, [iv], cz, mask=lane0)

            _fps_phase(px, py, pz, distv, _N1, _S1, store1)
            pltpu.sync_copy(sx, out_c1.at[pl.ds((b * 3 + 0) * _S1, _S1)])
            pltpu.sync_copy(sy, out_c1.at[pl.ds((b * 3 + 1) * _S1, _S1)])
            pltpu.sync_copy(sz, out_c1.at[pl.ds((b * 3 + 2) * _S1, _S1)])

            def store2(iv, lane0, cx, cy, cz):
                plsc.store_scatter(tx, [iv], cx, mask=lane0)
                plsc.store_scatter(ty, [iv], cy, mask=lane0)
                plsc.store_scatter(tz, [iv], cz, mask=lane0)
                plsc.store_scatter(stag4f, [iv * 4], cx, mask=lane0)
                plsc.store_scatter(stag4f, [iv * 4 + 1], cy, mask=lane0)
                plsc.store_scatter(stag4f, [iv * 4 + 2], cz, mask=lane0)
                plsc.store_scatter(stag4f, [iv * 4 + 3],
                                   jnp.zeros((16,), _F32), mask=lane0)

            _fps_phase(sx, sy, sz, distv, _S1, _S2, store2)
            pltpu.sync_copy(tx, out_c2.at[pl.ds((b * 3 + 0) * _S2, _S2)])
            pltpu.sync_copy(ty, out_c2.at[pl.ds((b * 3 + 1) * _S2, _S2)])
            pltpu.sync_copy(tz, out_c2.at[pl.ds((b * 3 + 2) * _S2, _S2)])
            pltpu.sync_copy(stag4f, out4f.at[pl.ds(b * _S2 * 4, _S2 * 4)])

        plsc.subcore_barrier()

        b = wid // 4
        part = wid % 4
        pltpu.sync_copy(pts_hbm.at[pl.ds((b * 3 + 0) * N, N)], px)
        pltpu.sync_copy(pts_hbm.at[pl.ds((b * 3 + 1) * N, N)], py)
        pltpu.sync_copy(pts_hbm.at[pl.ds((b * 3 + 2) * N, N)], pz)
        pltpu.sync_copy(out_c1.at[pl.ds((b * 3 + 0) * S + part * S4, S4)], cxb)
        pltpu.sync_copy(out_c1.at[pl.ds((b * 3 + 1) * S + part * S4, S4)], cyb)
        pltpu.sync_copy(out_c1.at[pl.ds((b * 3 + 2) * S + part * S4, S4)], czb)
        iot = lax.iota(_I32, 16)
        for j in range(CH):
            xc = px[pl.ds(j * 16, 16)]
            yc = py[pl.ds(j * 16, 16)]
            zc = pz[pl.ds(j * 16, 16)]
            pxb[pl.ds(j * 16, 16)] = _rne_bf16(xc)
            pyb[pl.ds(j * 16, 16)] = _rne_bf16(yc)
            pzb[pl.ds(j * 16, 16)] = _rne_bf16(zc)
            s2v[pl.ds(j * 16, 16)] = (xc * xc + yc * yc) + zc * zc

        def per_s(s, carry):
            sv = jnp.full((16,), s, _I32)
            cx = plsc.load_gather(cxb, [sv])
            cy = plsc.load_gather(cyb, [sv])
            cz = plsc.load_gather(czb, [sv])
            cbx = _rne_bf16(cx)
            cby = _rne_bf16(cy)
            cbz = _rne_bf16(cz)
            s1c = (cx * cx + cy * cy) + cz * cz
            cntv = jnp.zeros((16,), _I32)
            for j in range(CH):
                cross = (cbx * pxb[pl.ds(j * 16, 16)]
                         + cby * pyb[pl.ds(j * 16, 16)]) \
                    + cbz * pzb[pl.ds(j * 16, 16)]
                d = (s1c + s2v[pl.ds(j * 16, 16)]) - 2.0 * cross
                m = jnp.logical_not(d > r2)
                mi = m.astype(_I32)
                rank = plsc.cumsum(mi) - mi
                plsc.store_scatter(ib, [rank + cntv], iot + (j * 16), mask=m)
                cntv = cntv + plsc.all_reduce_population_count(m)
            lane0 = iot == 0
            i0 = jnp.sum(jnp.where(lane0, ib[pl.ds(0, 16)], 0))
            for kk in range(PK):
                kio = iot + kk * 16
                valid = kio < cntv
                oi = jnp.where(valid, ib[pl.ds(kk * 16, 16)], i0)
                ox = plsc.load_gather(px, [oi]) - cx
                oy = plsc.load_gather(py, [oi]) - cy
                oz = plsc.load_gather(pz, [oi]) - cz
                rows4 = (s * K + kio) * 4
                plsc.store_scatter(stag4b, [rows4], ox)
                plsc.store_scatter(stag4b, [rows4 + 1], oy)
                plsc.store_scatter(stag4b, [rows4 + 2], oz)
                plsc.store_scatter(stag4b, [rows4 + 3],
                                   jnp.zeros((16,), _F32))
            return carry

        lax.fori_loop(0, S4, per_s, jnp.int32(0))
        base = (b * S + part * S4) * K * 4
        pltpu.sync_copy(stag4b, out4b.at[pl.ds(base, S4 * K * 4)])

    return pl.kernel(body, out_type=out_type, mesh=mesh,
                     scratch_types=scratch,
                     compiler_params=pltpu.CompilerParams(
                         needs_layout_passes=False))


def _make_ballq(N, S, K, r2, gather_d):
    CH = N // 16
    S4 = S // 4
    PK = K // 16
    mesh = plsc.VectorSubcoreMesh(core_axis_name="c", subcore_axis_name="s")
    out_type = [jax.ShapeDtypeStruct((_B * S * K * 4,), _F32)]
    if gather_d:
        out_type.append(jax.ShapeDtypeStruct((_B * S * K, gather_d), _F32))
    scratch = [
        pltpu.VMEM((N,), _F32),
        pltpu.VMEM((N,), _F32),
        pltpu.VMEM((N,), _F32),
        pltpu.VMEM((N,), _F32),
        pltpu.VMEM((N,), _F32),
        pltpu.VMEM((N,), _F32),
        pltpu.VMEM((N,), _F32),
        pltpu.VMEM((S4,), _F32),
        pltpu.VMEM((S4,), _F32),
        pltpu.VMEM((S4,), _F32),
        pltpu.VMEM((N + 16,), _I32),
        pltpu.VMEM((S4 * K * 4,), _F32),
    ]
    if gather_d:
        scratch += [pltpu.VMEM((K,), _I32), pltpu.VMEM((K,), _I32),
                    pltpu.VMEM((K, gather_d), _F32),
                    pltpu.VMEM((K, gather_d), _F32),
                    pltpu.SemaphoreType.DMA, pltpu.SemaphoreType.DMA]

    def body(pts_hbm, ctr_hbm, *rest):
        if gather_d:
            (table_hbm, out4, outg, px, py, pz, pxb, pyb, pzb, s2v,
             cxb, cyb, czb, ib, stag4, idxr0, idxr1, gb0, gb1,
             sem0, sem1) = rest
        else:
            (out4, px, py, pz, pxb, pyb, pzb, s2v,
             cxb, cyb, czb, ib, stag4) = rest
        wid = lax.axis_index("s") * 2 + lax.axis_index("c")
        b = wid // 4
        part = wid % 4
        pltpu.sync_copy(pts_hbm.at[pl.ds((b * 3 + 0) * N, N)], px)
        pltpu.sync_copy(pts_hbm.at[pl.ds((b * 3 + 1) * N, N)], py)
        pltpu.sync_copy(pts_hbm.at[pl.ds((b * 3 + 2) * N, N)], pz)
        pltpu.sync_copy(ctr_hbm.at[pl.ds((b * 3 + 0) * S + part * S4, S4)], cxb)
        pltpu.sync_copy(ctr_hbm.at[pl.ds((b * 3 + 1) * S + part * S4, S4)], cyb)
        pltpu.sync_copy(ctr_hbm.at[pl.ds((b * 3 + 2) * S + part * S4, S4)], czb)
        iot = lax.iota(_I32, 16)
        for j in range(CH):
            xc = px[pl.ds(j * 16, 16)]
            yc = py[pl.ds(j * 16, 16)]
            zc = pz[pl.ds(j * 16, 16)]
            pxb[pl.ds(j * 16, 16)] = _rne_bf16(xc)
            pyb[pl.ds(j * 16, 16)] = _rne_bf16(yc)
            pzb[pl.ds(j * 16, 16)] = _rne_bf16(zc)
            s2v[pl.ds(j * 16, 16)] = (xc * xc + yc * yc) + zc * zc

        def centroid(s, idxr):
            sv = jnp.full((16,), s, _I32)
            cx = plsc.load_gather(cxb, [sv])
            cy = plsc.load_gather(cyb, [sv])
            cz = plsc.load_gather(czb, [sv])
            cbx = _rne_bf16(cx)
            cby = _rne_bf16(cy)
            cbz = _rne_bf16(cz)
            s1c = (cx * cx + cy * cy) + cz * cz
            cntv = jnp.zeros((16,), _I32)
            for j in range(CH):
                cross = (cbx * pxb[pl.ds(j * 16, 16)]
                         + cby * pyb[pl.ds(j * 16, 16)]) \
                    + cbz * pzb[pl.ds(j * 16, 16)]
                d = (s1c + s2v[pl.ds(j * 16, 16)]) - 2.0 * cross
                m = jnp.logical_not(d > r2)
                mi = m.astype(_I32)
                rank = plsc.cumsum(mi) - mi
                plsc.store_scatter(ib, [rank + cntv], iot + (j * 16), mask=m)
                cntv = cntv + plsc.all_reduce_population_count(m)
            lane0 = iot == 0
            i0 = jnp.sum(jnp.where(lane0, ib[pl.ds(0, 16)], 0))
            for kk in range(PK):
                kio = iot + kk * 16
                valid = kio < cntv
                oi = jnp.where(valid, ib[pl.ds(kk * 16, 16)], i0)
                ox = plsc.load_gather(px, [oi]) - cx
                oy = plsc.load_gather(py, [oi]) - cy
                oz = plsc.load_gather(pz, [oi]) - cz
                rows4 = (s * K + kio) * 4
                plsc.store_scatter(stag4, [rows4], ox)
                plsc.store_scatter(stag4, [rows4 + 1], oy)
                plsc.store_scatter(stag4, [rows4 + 2], oz)
                plsc.store_scatter(stag4, [rows4 + 3], jnp.zeros((16,), _F32))
                if gather_d:
                    idxr[pl.ds(kk * 16, 16)] = oi + b * N

        if gather_d:
            base_g = (b * S + part * S4) * K

            def per_pair(p, carry):
                s0 = p * 2
                centroid(s0, idxr0)
                d0 = pltpu.async_copy(table_hbm.at[idxr0], gb0, sem0)
                centroid(s0 + 1, idxr1)
                d1 = pltpu.async_copy(table_hbm.at[idxr1], gb1, sem1)
                d0.wait()
                pltpu.sync_copy(gb0, outg.at[pl.ds(base_g + s0 * K, K)])
                d1.wait()
                pltpu.sync_copy(gb1, outg.at[pl.ds(base_g + (s0 + 1) * K, K)])
                return carry

            lax.fori_loop(0, S4 // 2, per_pair, jnp.int32(0))
        else:
            def per_s(s, carry):
                centroid(s, None)
                return carry

            lax.fori_loop(0, S4, per_s, jnp.int32(0))
        base = (b * S + part * S4) * K * 4
        pltpu.sync_copy(stag4, out4.at[pl.ds(base, S4 * K * 4)])

    return pl.kernel(body, out_type=out_type, mesh=mesh,
                     scratch_types=scratch,
                     compiler_params=pltpu.CompilerParams(
                         needs_layout_passes=False))


def _dot(a, b):
    return jax.lax.dot_general(a, b, (((1,), (0,)), ((), ())),
                               preferred_element_type=_F32)


def _mlp1_body(x_ref, w1, b1, w2, b2, w3, b3, w4, b4, out_ref, *, SB, K):
    x = x_ref[...]
    h = jnp.maximum(_dot(x, w1[...]) + b1[...], 0.0)
    h = jnp.maximum(_dot(h, w2[...]) + b2[...], 0.0)
    h = jnp.maximum(_dot(h, w3[...]) + b3[...], 0.0)
    p = jnp.max(h.reshape(SB, K, h.shape[-1]), axis=1)
    out_ref[...] = _dot(p, w4[...]) + b4[...]


def _mlp2_body(x_ref, g_ref, w1x, w2, b2, w3, b3, out_ref, *, SB, K):
    x = x_ref[...]
    g = g_ref[...]
    h = jnp.maximum(g + _dot(x, w1x[...]), 0.0)
    h = jnp.maximum(_dot(h, w2[...]) + b2[...], 0.0)
    h = jnp.maximum(_dot(h, w3[...]) + b3[...], 0.0)
    out_ref[...] = jnp.max(h.reshape(SB, K, h.shape[-1]), axis=1)


def _head_body(x4_ref, pts_ref, w1x, w1p, b1, w2, b2, w3, b3,
               f1, fb1, f2, fb2, f3, fb3, logit_ref, pool_ref):
    x4 = x4_ref[...]
    pts = pts_ref[...]
    h = jnp.maximum(_dot(x4, w1x[...]) + _dot(pts, w1p[...]) + b1[...], 0.0)
    h = jnp.maximum(_dot(h, w2[...]) + b2[...], 0.0)
    h = jnp.maximum(_dot(h, w3[...]) + b3[...], 0.0)
    p = jnp.max(h.reshape(_B, _S2, 512), axis=1)
    y = jnp.maximum(_dot(p, f1[...]) + fb1[...], 0.0)
    y = jnp.maximum(_dot(y, f2[...]) + fb2[...], 0.0)
    logit_ref[...] = _dot(y, f3[...]) + fb3[...]
    pool_ref[...] = p


def _full(shape):
    return pl.BlockSpec(shape, lambda *_: tuple(0 for _ in shape))


def _fold(l):
    s = l["g"] / jnp.sqrt(1.0 + _EPS)
    return l["w"] * s[None, :], (l["b"] * s + l["be"]).reshape(1, -1)


def _pad4(w):
    return jnp.concatenate([w, jnp.zeros((1, w.shape[1]), _F32)], axis=0)


def kernel(xyz, params):
    sa1, sa2, sa3 = params["sa1"], params["sa2"], params["sa3"]
    w11, b11 = _fold(sa1[0])
    w12, b12 = _fold(sa1[1])
    w13, b13 = _fold(sa1[2])
    w21, b21 = _fold(sa2[0])
    w22, b22 = _fold(sa2[1])
    w23, b23 = _fold(sa2[2])
    w31, b31 = _fold(sa3[0])
    w32, b32 = _fold(sa3[1])
    w33, b33 = _fold(sa3[2])
    wf1, bf1 = _fold(params["fc1"])
    wf2, bf2 = _fold(params["fc2"])
    wf3 = params["fc3"]["w"]
    bf3 = params["fc3"]["b"].reshape(1, -1)
    w11p = _pad4(w11)
    w21x = _pad4(w21[:3])
    w21p = w21[3:]
    w31x = _pad4(w31[:3])
    w31p = w31[3:]

    xyzf = xyz.reshape(_B * 3 * _N1)
    newc1, newc2, new4_2, xyz4_1 = _make_stage1()(xyzf)
    xyz4_1 = xyz4_1.reshape(_B * _S1 * _K1, 4)

    SB1 = 128
    RB1 = SB1 * _K1
    t2 = pl.pallas_call(
        functools.partial(_mlp1_body, SB=SB1, K=_K1),
        grid=(_B * _S1 // SB1,),
        in_specs=[
            pl.BlockSpec((RB1, 4), lambda g: (g, 0)),
            _full(w11p.shape), _full(b11.shape),
            _full(w12.shape), _full(b12.shape),
            _full(w13.shape), _full(b13.shape),
            _full(w21p.shape), _full(b21.shape),
        ],
        out_specs=pl.BlockSpec((SB1, 128), lambda g: (g, 0)),
        out_shape=jax.ShapeDtypeStruct((_B * _S1, 128), _F32),
    )(xyz4_1, w11p, b11, w12, b12, w13, b13, w21p, b21)

    ballq2 = _make_ballq(_S1, _S2, _K2, _R2 * _R2, 128)
    xyz4_2, gath = ballq2(newc1, newc2, t2)
    xyz4_2 = xyz4_2.reshape(_B * _S2 * _K2, 4)

    SB2 = 64
    RB2 = SB2 * _K2
    l2p = pl.pallas_call(
        functools.partial(_mlp2_body, SB=SB2, K=_K2),
        grid=(_B * _S2 // SB2,),
        in_specs=[
            pl.BlockSpec((RB2, 4), lambda g: (g, 0)),
            pl.BlockSpec((RB2, 128), lambda g: (g, 0)),
            _full(w21x.shape),
            _full(w22.shape), _full(b22.shape),
            _full(w23.shape), _full(b23.shape),
        ],
        out_specs=pl.BlockSpec((SB2, 256), lambda g: (g, 0)),
        out_shape=jax.ShapeDtypeStruct((_B * _S2, 256), _F32),
    )(xyz4_2, gath, w21x, w22, b22, w23, b23)

    logits, pooled = pl.pallas_call(
        _head_body,
        out_shape=[
            jax.ShapeDtypeStruct((_B, 40), _F32),
            jax.ShapeDtypeStruct((_B, 512), _F32),
        ],
    )(new4_2.reshape(_B * _S2, 4), l2p,
      w31x, w31p, b31, w32, b32, w33, b33,
      wf1, bf1, wf2, bf2, wf3, bf3)

    return logits, pooled.reshape(_B, 512, 1)


# speedup vs baseline: 1.0074x; 1.0074x over previous
import functools

import jax
import jax.numpy as jnp
from jax import lax
from jax.experimental import pallas as pl
from jax.experimental.pallas import tpu as pltpu
from jax.experimental.pallas import tpu_sc as plsc

_B = 8
_N1 = 1024
_S1 = 512
_K1 = 32
_R1 = 0.2
_S2 = 256
_K2 = 64
_R2 = 0.4
_H = 64
_EPS = 1e-5
_F32 = jnp.float32
_I32 = jnp.int32


def _rne_bf16(x):
    bits = plsc.bitcast(x, _I32)
    half = jnp.full((16,), 0x7FFF, _I32)
    one = jnp.full((16,), 1, _I32)
    lsb = lax.shift_right_logical(bits, jnp.full((16,), 16, _I32)) & one
    r = (bits + half + lsb) & jnp.full((16,), -65536, _I32)
    return plsc.bitcast(r, _F32)


def _fps_phase(px, py, pz, distv, N, S, store_cb):
    CH = N // 16
    for j in range(CH):
        distv[pl.ds(j * 16, 16)] = jnp.full((16,), 1e10, _F32)
    iot = lax.iota(_I32, 16)
    lane0 = iot == 0

    def step(i, f):
        fv = jnp.full((16,), f, _I32)
        iv = jnp.full((16,), i, _I32)
        cx = plsc.load_gather(px, [fv])
        cy = plsc.load_gather(py, [fv])
        cz = plsc.load_gather(pz, [fv])
        store_cb(iv, lane0, cx, cy, cz)
        bestv = jnp.full((16,), -1.0, _F32)
        besti = jnp.zeros((16,), _I32)
        for j in range(CH):
            dx = px[pl.ds(j * 16, 16)] - cx
            dy = py[pl.ds(j * 16, 16)] - cy
            dz = pz[pl.ds(j * 16, 16)] - cz
            d = (dx * dx + dy * dy) + dz * dz
            dn = jnp.minimum(distv[pl.ds(j * 16, 16)], d)
            distv[pl.ds(j * 16, 16)] = dn
            upd = dn > bestv
            bestv = jnp.where(upd, dn, bestv)
            besti = jnp.where(upd, iot + (j * 16), besti)
        m = jnp.max(bestv)
        cand = jnp.where(bestv == m, besti, N)
        return jnp.min(cand)

    lax.fori_loop(0, S, step, jnp.int32(0))


def _make_stage1():
    N, S, K, r2 = _N1, _S1, _K1, _R1 * _R1
    CH = N // 16
    S4 = S // 4
    PK = K // 16
    mesh = plsc.VectorSubcoreMesh(core_axis_name="c", subcore_axis_name="s")
    out_type = [
        jax.ShapeDtypeStruct((_B * 3 * _S1,), _F32),
        jax.ShapeDtypeStruct((_B * 3 * _S2,), _F32),
        jax.ShapeDtypeStruct((_B * _S2 * 4,), _F32),
        jax.ShapeDtypeStruct((_B * S * K * 4,), _F32),
    ]
    scratch = [
        pltpu.VMEM((N,), _F32),
        pltpu.VMEM((N,), _F32),
        pltpu.VMEM((N,), _F32),
        pltpu.VMEM((N,), _F32),
        pltpu.VMEM((_S1,), _F32),
        pltpu.VMEM((_S1,), _F32),
        pltpu.VMEM((_S1,), _F32),
        pltpu.VMEM((_S2,), _F32),
        pltpu.VMEM((_S2,), _F32),
        pltpu.VMEM((_S2,), _F32),
        pltpu.VMEM((_S2 * 4,), _F32),
        pltpu.VMEM((N,), _F32),
        pltpu.VMEM((N,), _F32),
        pltpu.VMEM((N,), _F32),
        pltpu.VMEM((N,), _F32),
        pltpu.VMEM((S4,), _F32),
        pltpu.VMEM((S4,), _F32),
        pltpu.VMEM((S4,), _F32),
        pltpu.VMEM((N + 16,), _I32),
        pltpu.VMEM((S4 * K * 4,), _F32),
    ]

    def body(pts_hbm, out_c1, out_c2, out4f, out4b,
             px, py, pz, distv, sx, sy, sz, tx, ty, tz, stag4f,
             pxb, pyb, pzb, s2v, cxb, cyb, czb, ib, stag4b):
        wid = lax.axis_index("c") * 16 + lax.axis_index("s")

        @pl.when(wid % 4 == 0)
        def _():
            b = wid // 4
            pltpu.sync_copy(pts_hbm.at[pl.ds((b * 3 + 0) * N, N)], px)
            pltpu.sync_copy(pts_hbm.at[pl.ds((b * 3 + 1) * N, N)], py)
            pltpu.sync_copy(pts_hbm.at[pl.ds((b * 3 + 2) * N, N)], pz)

            def store1(iv, lane0, cx, cy, cz):
                plsc.store_scatter(sx, [iv], cx, mask=lane0)
                plsc.store_scatter(sy, [iv], cy, mask=lane0)
                plsc.store_scatter(sz, [iv], cz, mask=lane0)

            _fps_phase(px, py, pz, distv, _N1, _S1, store1)
            pltpu.sync_copy(sx, out_c1.at[pl.ds((b * 3 + 0) * _S1, _S1)])
            pltpu.sync_copy(sy, out_c1.at[pl.ds((b * 3 + 1) * _S1, _S1)])
            pltpu.sync_copy(sz, out_c1.at[pl.ds((b * 3 + 2) * _S1, _S1)])

            def store2(iv, lane0, cx, cy, cz):
                plsc.store_scatter(tx, [iv], cx, mask=lane0)
                plsc.store_scatter(ty, [iv], cy, mask=lane0)
                plsc.store_scatter(tz, [iv], cz, mask=lane0)
                plsc.store_scatter(stag4f, [iv * 4], cx, mask=lane0)
                plsc.store_scatter(stag4f, [iv * 4 + 1], cy, mask=lane0)
                plsc.store_scatter(stag4f, [iv * 4 + 2], cz, mask=lane0)
                plsc.store_scatter(stag4f, [iv * 4 + 3],
                                   jnp.zeros((16,), _F32), mask=lane0)

            _fps_phase(sx, sy, sz, distv, _S1, _S2, store2)
            pltpu.sync_copy(tx, out_c2.at[pl.ds((b * 3 + 0) * _S2, _S2)])
            pltpu.sync_copy(ty, out_c2.at[pl.ds((b * 3 + 1) * _S2, _S2)])
            pltpu.sync_copy(tz, out_c2.at[pl.ds((b * 3 + 2) * _S2, _S2)])
            pltpu.sync_copy(stag4f, out4f.at[pl.ds(b * _S2 * 4, _S2 * 4)])

        plsc.subcore_barrier()

        b = wid // 4
        part = wid % 4
        pltpu.sync_copy(pts_hbm.at[pl.ds((b * 3 + 0) * N, N)], px)
        pltpu.sync_copy(pts_hbm.at[pl.ds((b * 3 + 1) * N, N)], py)
        pltpu.sync_copy(pts_hbm.at[pl.ds((b * 3 + 2) * N, N)], pz)
        pltpu.sync_copy(out_c1.at[pl.ds((b * 3 + 0) * S + part * S4, S4)], cxb)
        pltpu.sync_copy(out_c1.at[pl.ds((b * 3 + 1) * S + part * S4, S4)], cyb)
        pltpu.sync_copy(out_c1.at[pl.ds((b * 3 + 2) * S + part * S4, S4)], czb)
        iot = lax.iota(_I32, 16)
        for j in range(CH):
            xc = px[pl.ds(j * 16, 16)]
            yc = py[pl.ds(j * 16, 16)]
            zc = pz[pl.ds(j * 16, 16)]
            pxb[pl.ds(j * 16, 16)] = _rne_bf16(xc)
            pyb[pl.ds(j * 16, 16)] = _rne_bf16(yc)
            pzb[pl.ds(j * 16, 16)] = _rne_bf16(zc)
            s2v[pl.ds(j * 16, 16)] = (xc * xc + yc * yc) + zc * zc

        def per_s(s, carry):
            sv = jnp.full((16,), s, _I32)
            cx = plsc.load_gather(cxb, [sv])
            cy = plsc.load_gather(cyb, [sv])
            cz = plsc.load_gather(czb, [sv])
            cbx = _rne_bf16(cx)
            cby = _rne_bf16(cy)
            cbz = _rne_bf16(cz)
            s1c = (cx * cx + cy * cy) + cz * cz
            cntv = jnp.zeros((16,), _I32)
            for j in range(CH):
                cross = (cbx * pxb[pl.ds(j * 16, 16)]
                         + cby * pyb[pl.ds(j * 16, 16)]) \
                    + cbz * pzb[pl.ds(j * 16, 16)]
                d = (s1c + s2v[pl.ds(j * 16, 16)]) - 2.0 * cross
                m = jnp.logical_not(d > r2)
                mi = m.astype(_I32)
                rank = plsc.cumsum(mi) - mi
                plsc.store_scatter(ib, [rank + cntv], iot + (j * 16), mask=m)
                cntv = cntv + plsc.all_reduce_population_count(m)
            lane0 = iot == 0
            i0 = jnp.sum(jnp.where(lane0, ib[pl.ds(0, 16)], 0))
            for kk in range(PK):
                kio = iot + kk * 16
                valid = kio < cntv
                oi = jnp.where(valid, ib[pl.ds(kk * 16, 16)], i0)
                ox = plsc.load_gather(px, [oi]) - cx
                oy = plsc.load_gather(py, [oi]) - cy
                oz = plsc.load_gather(pz, [oi]) - cz
                rows4 = (s * K + kio) * 4
                plsc.store_scatter(stag4b, [rows4], ox)
                plsc.store_scatter(stag4b, [rows4 + 1], oy)
                plsc.store_scatter(stag4b, [rows4 + 2], oz)
                plsc.store_scatter(stag4b, [rows4 + 3],
                                   jnp.zeros((16,), _F32))
            return carry

        lax.fori_loop(0, S4, per_s, jnp.int32(0))
        base = (b * S + part * S4) * K * 4
        pltpu.sync_copy(stag4b, out4b.at[pl.ds(base, S4 * K * 4)])

    return pl.kernel(body, out_type=out_type, mesh=mesh,
                     scratch_types=scratch,
                     compiler_params=pltpu.CompilerParams(
                         needs_layout_passes=False))


def _make_ballq(N, S, K, r2, gather_d):
    CH = N // 16
    S4 = S // 4
    PK = K // 16
    mesh = plsc.VectorSubcoreMesh(core_axis_name="c", subcore_axis_name="s")
    out_type = [jax.ShapeDtypeStruct((_B * S * K * 4,), _F32)]
    if gather_d:
        out_type.append(jax.ShapeDtypeStruct((_B * S * K, gather_d), _F32))
    scratch = [
        pltpu.VMEM((N,), _F32),
        pltpu.VMEM((N,), _F32),
        pltpu.VMEM((N,), _F32),
        pltpu.VMEM((N,), _F32),
        pltpu.VMEM((N,), _F32),
        pltpu.VMEM((N,), _F32),
        pltpu.VMEM((N,), _F32),
        pltpu.VMEM((S4,), _F32),
        pltpu.VMEM((S4,), _F32),
        pltpu.VMEM((S4,), _F32),
        pltpu.VMEM((N + 16,), _I32),
        pltpu.VMEM((S4 * K * 4,), _F32),
    ]
    if gather_d:
        scratch += [pltpu.VMEM((K,), _I32), pltpu.VMEM((K,), _I32),
                    pltpu.VMEM((K, gather_d), _F32),
                    pltpu.VMEM((K, gather_d), _F32),
                    pltpu.SemaphoreType.DMA, pltpu.SemaphoreType.DMA]

    def body(pts_hbm, ctr_hbm, *rest):
        if gather_d:
            (table_hbm, out4, outg, px, py, pz, pxb, pyb, pzb, s2v,
             cxb, cyb, czb, ib, stag4, idxr0, idxr1, gb0, gb1,
             sem0, sem1) = rest
        else:
            (out4, px, py, pz, pxb, pyb, pzb, s2v,
             cxb, cyb, czb, ib, stag4) = rest
        wid = lax.axis_index("s") * 2 + lax.axis_index("c")
        b = wid // 4
        part = wid % 4
        pltpu.sync_copy(pts_hbm.at[pl.ds((b * 3 + 0) * N, N)], px)
        pltpu.sync_copy(pts_hbm.at[pl.ds((b * 3 + 1) * N, N)], py)
        pltpu.sync_copy(pts_hbm.at[pl.ds((b * 3 + 2) * N, N)], pz)
        pltpu.sync_copy(ctr_hbm.at[pl.ds((b * 3 + 0) * S + part * S4, S4)], cxb)
        pltpu.sync_copy(ctr_hbm.at[pl.ds((b * 3 + 1) * S + part * S4, S4)], cyb)
        pltpu.sync_copy(ctr_hbm.at[pl.ds((b * 3 + 2) * S + part * S4, S4)], czb)
        iot = lax.iota(_I32, 16)
        for j in range(CH):
            xc = px[pl.ds(j * 16, 16)]
            yc = py[pl.ds(j * 16, 16)]
            zc = pz[pl.ds(j * 16, 16)]
            pxb[pl.ds(j * 16, 16)] = _rne_bf16(xc)
            pyb[pl.ds(j * 16, 16)] = _rne_bf16(yc)
            pzb[pl.ds(j * 16, 16)] = _rne_bf16(zc)
            s2v[pl.ds(j * 16, 16)] = (xc * xc + yc * yc) + zc * zc

        def centroid(s, idxr):
            sv = jnp.full((16,), s, _I32)
            cx = plsc.load_gather(cxb, [sv])
            cy = plsc.load_gather(cyb, [sv])
            cz = plsc.load_gather(czb, [sv])
            cbx = _rne_bf16(cx)
            cby = _rne_bf16(cy)
            cbz = _rne_bf16(cz)
            s1c = (cx * cx + cy * cy) + cz * cz
            cntv = jnp.zeros((16,), _I32)
            for j in range(CH):
                cross = (cbx * pxb[pl.ds(j * 16, 16)]
                         + cby * pyb[pl.ds(j * 16, 16)]) \
                    + cbz * pzb[pl.ds(j * 16, 16)]
                d = (s1c + s2v[pl.ds(j * 16, 16)]) - 2.0 * cross
                m = jnp.logical_not(d > r2)
                mi = m.astype(_I32)
                rank = plsc.cumsum(mi) - mi
                plsc.store_scatter(ib, [rank + cntv], iot + (j * 16), mask=m)
                cntv = cntv + plsc.all_reduce_population_count(m)
            lane0 = iot == 0
            i0 = jnp.sum(jnp.where(lane0, ib[pl.ds(0, 16)], 0))
            for kk in range(PK):
                kio = iot + kk * 16
                valid = kio < cntv
                oi = jnp.where(valid, ib[pl.ds(kk * 16, 16)], i0)
                ox = plsc.load_gather(px, [oi]) - cx
                oy = plsc.load_gather(py, [oi]) - cy
                oz = plsc.load_gather(pz, [oi]) - cz
                rows4 = (s * K + kio) * 4
                plsc.store_scatter(stag4, [rows4], ox)
                plsc.store_scatter(stag4, [rows4 + 1], oy)
                plsc.store_scatter(stag4, [rows4 + 2], oz)
                plsc.store_scatter(stag4, [rows4 + 3], jnp.zeros((16,), _F32))
                if gather_d:
                    idxr[pl.ds(kk * 16, 16)] = oi + b * N

        if gather_d:
            base_g = (b * S + part * S4) * K

            def per_pair(p, carry):
                s0 = p * 2
                centroid(s0, idxr0)
                d0 = pltpu.async_copy(table_hbm.at[idxr0], gb0, sem0)
                centroid(s0 + 1, idxr1)
                d1 = pltpu.async_copy(table_hbm.at[idxr1], gb1, sem1)
                d0.wait()
                pltpu.sync_copy(gb0, outg.at[pl.ds(base_g + s0 * K, K)])
                d1.wait()
                pltpu.sync_copy(gb1, outg.at[pl.ds(base_g + (s0 + 1) * K, K)])
                return carry

            lax.fori_loop(0, S4 // 2, per_pair, jnp.int32(0))
        else:
            def per_s(s, carry):
                centroid(s, None)
                return carry

            lax.fori_loop(0, S4, per_s, jnp.int32(0))
        base = (b * S + part * S4) * K * 4
        pltpu.sync_copy(stag4, out4.at[pl.ds(base, S4 * K * 4)])

    return pl.kernel(body, out_type=out_type, mesh=mesh,
                     scratch_types=scratch,
                     compiler_params=pltpu.CompilerParams(
                         needs_layout_passes=False))


def _dot(a, b):
    return jax.lax.dot_general(a, b, (((1,), (0,)), ((), ())),
                               preferred_element_type=_F32)


def _mlp1_body(x_ref, w1, b1, w2, b2, w3, b3, w4, b4, out_ref, *, SB, K):
    x = x_ref[...]
    h = jnp.maximum(_dot(x, w1[...]) + b1[...], 0.0)
    h = jnp.maximum(_dot(h, w2[...]) + b2[...], 0.0)
    h = jnp.maximum(_dot(h, w3[...]) + b3[...], 0.0)
    p = jnp.max(h.reshape(SB, K, h.shape[-1]), axis=1)
    out_ref[...] = _dot(p, w4[...]) + b4[...]


def _mlp2head_body(x_ref, g_ref, x4_ref, w1x, w2, b2, w3, b3,
                   hw1x, hw1p, hb1, hw2, hb2, hw3, hb3,
                   f1, fb1, f2, fb2, f3, fb3,
                   logit_ref, pool_ref, acc_ref, *, SB, K, NSTEP):
    gi = pl.program_id(0)
    x = x_ref[...]
    g = g_ref[...]
    h = jnp.maximum(g + _dot(x, w1x[...]), 0.0)
    h = jnp.maximum(_dot(h, w2[...]) + b2[...], 0.0)
    h = jnp.maximum(_dot(h, w3[...]) + b3[...], 0.0)
    acc_ref[pl.ds(gi * SB, SB), :] = jnp.max(
        h.reshape(SB, K, h.shape[-1]), axis=1)

    @pl.when(gi == NSTEP - 1)
    def _():
        x4 = x4_ref[...]
        pts = acc_ref[...]
        hh = jnp.maximum(
            _dot(x4, hw1x[...]) + _dot(pts, hw1p[...]) + hb1[...], 0.0)
        hh = jnp.maximum(_dot(hh, hw2[...]) + hb2[...], 0.0)
        hh = jnp.maximum(_dot(hh, hw3[...]) + hb3[...], 0.0)
        p = jnp.max(hh.reshape(_B, _S2, 512), axis=1)
        y = jnp.maximum(_dot(p, f1[...]) + fb1[...], 0.0)
        y = jnp.maximum(_dot(y, f2[...]) + fb2[...], 0.0)
        logit_ref[...] = _dot(y, f3[...]) + fb3[...]
        pool_ref[...] = p


def _full(shape):
    return pl.BlockSpec(shape, lambda *_: tuple(0 for _ in shape))


def _fold(l):
    s = l["g"] / jnp.sqrt(1.0 + _EPS)
    return l["w"] * s[None, :], (l["b"] * s + l["be"]).reshape(1, -1)


def _pad4(w):
    return jnp.concatenate([w, jnp.zeros((1, w.shape[1]), _F32)], axis=0)


def kernel(xyz, params):
    sa1, sa2, sa3 = params["sa1"], params["sa2"], params["sa3"]
    w11, b11 = _fold(sa1[0])
    w12, b12 = _fold(sa1[1])
    w13, b13 = _fold(sa1[2])
    w21, b21 = _fold(sa2[0])
    w22, b22 = _fold(sa2[1])
    w23, b23 = _fold(sa2[2])
    w31, b31 = _fold(sa3[0])
    w32, b32 = _fold(sa3[1])
    w33, b33 = _fold(sa3[2])
    wf1, bf1 = _fold(params["fc1"])
    wf2, bf2 = _fold(params["fc2"])
    wf3 = params["fc3"]["w"]
    bf3 = params["fc3"]["b"].reshape(1, -1)
    w11p = _pad4(w11)
    w21x = _pad4(w21[:3])
    w21p = w21[3:]
    w31x = _pad4(w31[:3])
    w31p = w31[3:]

    xyzf = xyz.reshape(_B * 3 * _N1)
    newc1, newc2, new4_2, xyz4_1 = _make_stage1()(xyzf)
    xyz4_1 = xyz4_1.reshape(_B * _S1 * _K1, 4)

    SB1 = 128
    RB1 = SB1 * _K1
    t2 = pl.pallas_call(
        functools.partial(_mlp1_body, SB=SB1, K=_K1),
        grid=(_B * _S1 // SB1,),
        in_specs=[
            pl.BlockSpec((RB1, 4), lambda g: (g, 0)),
            _full(w11p.shape), _full(b11.shape),
            _full(w12.shape), _full(b12.shape),
            _full(w13.shape), _full(b13.shape),
            _full(w21p.shape), _full(b21.shape),
        ],
        out_specs=pl.BlockSpec((SB1, 128), lambda g: (g, 0)),
        out_shape=jax.ShapeDtypeStruct((_B * _S1, 128), _F32),
    )(xyz4_1, w11p, b11, w12, b12, w13, b13, w21p, b21)

    ballq2 = _make_ballq(_S1, _S2, _K2, _R2 * _R2, 128)
    xyz4_2, gath = ballq2(newc1, newc2, t2)
    xyz4_2 = xyz4_2.reshape(_B * _S2 * _K2, 4)

    SB2 = 64
    RB2 = SB2 * _K2
    NSTEP = _B * _S2 // SB2
    logits, pooled = pl.pallas_call(
        functools.partial(_mlp2head_body, SB=SB2, K=_K2, NSTEP=NSTEP),
        grid=(NSTEP,),
        in_specs=[
            pl.BlockSpec((RB2, 4), lambda g: (g, 0)),
            pl.BlockSpec((RB2, 128), lambda g: (g, 0)),
            _full((_B * _S2, 4)),
            _full(w21x.shape),
            _full(w22.shape), _full(b22.shape),
            _full(w23.shape), _full(b23.shape),
            _full(w31x.shape), _full(w31p.shape), _full(b31.shape),
            _full(w32.shape), _full(b32.shape),
            _full(w33.shape), _full(b33.shape),
            _full(wf1.shape), _full(bf1.shape),
            _full(wf2.shape), _full(bf2.shape),
            _full(wf3.shape), _full(bf3.shape),
        ],
        out_specs=[
            pl.BlockSpec((_B, 40), lambda g: (0, 0)),
            pl.BlockSpec((_B, 512), lambda g: (0, 0)),
        ],
        out_shape=[
            jax.ShapeDtypeStruct((_B, 40), _F32),
            jax.ShapeDtypeStruct((_B, 512), _F32),
        ],
        scratch_shapes=[pltpu.VMEM((_B * _S2, 256), _F32)],
    )(xyz4_2, gath, new4_2.reshape(_B * _S2, 4),
      w21x, w22, b22, w23, b23,
      w31x, w31p, b31, w32, b32, w33, b33,
      wf1, bf1, wf2, bf2, wf3, bf3)

    return logits, pooled.reshape(_B, 512, 1)
